# baseline (speedup 1.0000x reference)
import functools

import jax
import jax.numpy as jnp
from jax import lax
from jax.experimental import pallas as pl
from jax.experimental.pallas import tpu as pltpu
from jax.experimental.pallas import tpu_sc as plsc

CHUNK = 16
UNROLL = 8


def kernel(x, pe):
    batch, seq_len, dim = x.shape
    info = plsc.get_sparse_core_info()
    n_workers = info.num_cores * info.num_subcores
    s_per_w = seq_len // n_workers
    nchunks = s_per_w // CHUNK
    cw = CHUNK * dim
    mesh = plsc.VectorSubcoreMesh(core_axis_name="c", subcore_axis_name="s")

    @functools.partial(
        pl.kernel,
        mesh=mesh,
        out_type=jax.ShapeDtypeStruct((batch * seq_len * dim,), x.dtype),
        scratch_types=[
            pltpu.VMEM((cw,), jnp.float32),
            pltpu.VMEM((cw,), jnp.float32),
            pltpu.VMEM((cw,), jnp.float32),
            pltpu.VMEM((cw,), jnp.float32),
            pltpu.VMEM((cw,), jnp.float32),
            pltpu.VMEM((cw,), jnp.float32),
            pltpu.SemaphoreType.DMA,
            pltpu.SemaphoreType.DMA,
            pltpu.SemaphoreType.DMA,
            pltpu.SemaphoreType.DMA,
            pltpu.SemaphoreType.DMA,
            pltpu.SemaphoreType.DMA,
        ],
    )
    def sc_add(x_hbm, pe_hbm, out_hbm, xin0, xin1, xo0, xo1, pb0, pb1,
               sxi0, sxi1, sxo0, sxo1, sp0, sp1):
        wid = lax.axis_index("s") * info.num_cores + lax.axis_index("c")
        fbase = wid * s_per_w * dim

        xin = (xin0, xin1)
        xo = (xo0, xo1)
        pb = (pb0, pb1)
        sxi = (sxi0, sxi1)
        sxo = (sxo0, sxo1)
        sp = (sp0, sp1)

        def x_off(c, b):
            return b * (seq_len * dim) + fbase + c * cw

        def issue_in(p, c, b):
            pltpu.async_copy(x_hbm.at[pl.ds(x_off(c, b), cw)], xin[p], sxi[p])

        def wait_in(p):
            pltpu.make_async_copy(
                x_hbm.at[pl.ds(0, cw)], xin[p], sxi[p]).wait()

        def issue_out(p, c, b):
            pltpu.async_copy(xo[p], out_hbm.at[pl.ds(x_off(c, b), cw)], sxo[p])

        def wait_out(p):
            pltpu.make_async_copy(
                xo[p], out_hbm.at[pl.ds(0, cw)], sxo[p]).wait()

        def issue_pe(q, c):
            pltpu.async_copy(pe_hbm.at[pl.ds(fbase + c * cw, cw)], pb[q], sp[q])

        def wait_pe(q):
            pltpu.make_async_copy(
                pe_hbm.at[pl.ds(0, cw)], pb[q], sp[q]).wait()

        def add_chunk(p, q):
            @plsc.parallel_loop(0, cw, step=16, unroll=UNROLL)
            def _(i):
                sl = pl.ds(i, 16)
                xo[p][sl] = xin[p][sl] + pb[q][sl]

        issue_in(0, 0, 0)
        issue_in(1, 0, 1)
        issue_pe(0, 0)
        issue_pe(1, 1)

        def chunk_pair(cc, carry):
            for j in range(2):
                c = cc * 2 + j
                wait_pe(j)
                for b in range(batch):
                    p = b % 2
                    wait_in(p)
                    if b >= 2:
                        wait_out(p)
                    else:
                        @pl.when(c > 0)
                        def _():
                            wait_out(p)
                    add_chunk(p, j)
                    issue_out(p, c, b)
                    if b < 2:
                        issue_in(p, c, b + 2)
                    else:
                        @pl.when(c < nchunks - 1)
                        def _():
                            issue_in(p, c + 1, b - 2)

                @pl.when(c + 2 < nchunks)
                def _():
                    issue_pe(j, c + 2)
            return carry

        lax.fori_loop(0, nchunks // 2, chunk_pair, 0)
        wait_out(0)
        wait_out(1)

    out = sc_add(x.reshape(-1), pe[:seq_len].reshape(-1))
    return out.reshape(batch, seq_len, dim)
